# baseline (speedup 1.0000x reference)
import jax
import jax.numpy as jnp
from jax import lax
from jax.experimental import pallas as pl
from jax.experimental.pallas import tpu as pltpu
from jax.experimental.pallas import tpu_sc as plsc

_FIELD_DIM = 100000
_B = 16384
_F = 26
_TOTAL = _FIELD_DIM * _F

_NC = 2
_NS = 16
_NW = _NC * _NS
_BPW = _B // _NW
_EPW = _BPW * _F
_LANES = 16

_CHUNK = 64
_NCHUNK = _EPW // _CHUNK
_DEPTH = 16


def _body(tab2_hbm, x2_hbm, w_hbm, out_hbm,
          xv2, wv, idxv, embv, outv, sem, wsem,
          gsem0, gsem1, gsem2, gsem3):
    wid = lax.axis_index("s") * _NC + lax.axis_index("c")
    bbase = wid * _BPW

    cp_x = pltpu.async_copy(x2_hbm.at[:, pl.ds(bbase, _BPW)], xv2, sem)
    wcps = [pltpu.async_copy(w_hbm.at[pl.ds(f * _B + bbase, _BPW)],
                             wv.at[pl.ds(f * _BPW, _BPW)], wsem)
            for f in range(_F)]
    cp_x.wait()

    t1d = tab2_hbm.at[0]
    bsems = (gsem0, gsem1, gsem2, gsem3)

    def idx_block(c):
        def idx_j(j, _):
            o = c * _CHUNK + j * _LANES
            for f in range(_F):
                off = jnp.int32(f * _FIELD_DIM)
                idxv[pl.ds(f * _BPW + o, _LANES)] = (
                    xv2[f, pl.ds(o, _LANES)] + off
                )
            return 0
        lax.fori_loop(0, _CHUNK // _LANES, idx_j, 0)

    def fire_block(c):
        for f in range(_F):
            off = f * _BPW + c * _CHUNK
            pltpu.async_copy(
                t1d.at[idxv.at[pl.ds(off, _CHUNK)]],
                embv.at[pl.ds(off, _CHUNK)], bsems[c % len(bsems)])

    def drain_block(c):
        for f in range(_F):
            pltpu.make_async_copy(
                t1d.at[idxv.at[pl.ds(0, _CHUNK)]],
                embv.at[pl.ds(0, _CHUNK)], bsems[c % len(bsems)]).wait()

    def reduce_block(c):
        def red_g(g, _):
            rbase = c * _CHUNK + g * _LANES
            acc = jnp.zeros((_LANES,), jnp.float32)
            for f in range(_F):
                o = f * _BPW + rbase
                acc = acc + embv[pl.ds(o, _LANES)] * wv[pl.ds(o, _LANES)]
            outv[pl.ds(rbase, _LANES)] = acc
            return 0
        lax.fori_loop(0, _CHUNK // _LANES, red_g, 0)

    nblk = _BPW // _CHUNK
    _AHEAD = 4
    for c in range(_AHEAD):
        idx_block(c)
        fire_block(c)
    for cp in wcps:
        cp.wait()
    for c in range(_AHEAD, nblk):
        idx_block(c)
        fire_block(c)
        drain_block(c - _AHEAD)
        reduce_block(c - _AHEAD)
    for c in range(nblk - _AHEAD, nblk):
        drain_block(c)
        reduce_block(c)

    pltpu.sync_copy(outv, out_hbm.at[pl.ds(bbase, _BPW)])


_MESH = plsc.VectorSubcoreMesh(core_axis_name="c", subcore_axis_name="s")


@jax.jit
def _sc_lookup(table2d, x2d, w_t):
    f = pl.kernel(
        _body,
        out_type=jax.ShapeDtypeStruct((_B,), jnp.float32),
        mesh=_MESH,
        scratch_types=[
            pltpu.VMEM((_F, _BPW), jnp.int32),
            pltpu.VMEM((_EPW,), jnp.float32),
            pltpu.VMEM((_EPW,), jnp.int32),
            pltpu.VMEM((_EPW,), jnp.float32),
            pltpu.VMEM((_BPW,), jnp.float32),
            pltpu.SemaphoreType.DMA,
            pltpu.SemaphoreType.DMA,
            pltpu.SemaphoreType.DMA,
            pltpu.SemaphoreType.DMA,
            pltpu.SemaphoreType.DMA,
            pltpu.SemaphoreType.DMA,
        ],
    )
    return f(table2d, x2d, w_t)


def kernel(x, weight, fc_table, bias):
    w_t = jnp.transpose(weight, (1, 2, 0)).reshape(-1)
    out = _sc_lookup(fc_table.T, x.T, w_t)
    return out[:, None] + bias[None, :]


# speedup vs baseline: 1.1108x; 1.1108x over previous
import jax
import jax.numpy as jnp
from jax import lax
from jax.experimental import pallas as pl
from jax.experimental.pallas import tpu as pltpu
from jax.experimental.pallas import tpu_sc as plsc

_FIELD_DIM = 100000
_B = 16384
_F = 26
_TOTAL = _FIELD_DIM * _F

_NC = 2
_NS = 16
_NW = _NC * _NS
_BPW = _B // _NW
_EPW = _BPW * _F
_LANES = 16

_CHUNK = 128
_NCHUNK = _EPW // _CHUNK


def _body(tab2_hbm, x2_hbm, w_hbm, out_hbm,
          xv2, wv, idxv, embv, outv, sem, wsem,
          gsem0, gsem1, gsem2, gsem3):
    wid = lax.axis_index("s") * _NC + lax.axis_index("c")
    bbase = wid * _BPW

    cp_x = pltpu.async_copy(x2_hbm.at[:, pl.ds(bbase, _BPW)], xv2, sem)
    wcps = [pltpu.async_copy(w_hbm.at[pl.ds(f * _B + bbase, _BPW)],
                             wv.at[pl.ds(f * _BPW, _BPW)], wsem)
            for f in range(_F)]
    cp_x.wait()

    t1d = tab2_hbm.at[0]
    bsems = (gsem0, gsem1, gsem2, gsem3)

    def idx_block(c):
        def idx_j(j, _):
            o = c * _CHUNK + j * _LANES
            for f in range(_F):
                off = jnp.int32(f * _FIELD_DIM)
                idxv[pl.ds(f * _BPW + o, _LANES)] = (
                    xv2[f, pl.ds(o, _LANES)] + off
                )
            return 0
        lax.fori_loop(0, _CHUNK // _LANES, idx_j, 0)

    def fire_block(c):
        for f in range(_F):
            off = f * _BPW + c * _CHUNK
            pltpu.async_copy(
                t1d.at[idxv.at[pl.ds(off, _CHUNK)]],
                embv.at[pl.ds(off, _CHUNK)], bsems[c % len(bsems)])

    def drain_block(c):
        for f in range(_F):
            pltpu.make_async_copy(
                t1d.at[idxv.at[pl.ds(0, _CHUNK)]],
                embv.at[pl.ds(0, _CHUNK)], bsems[c % len(bsems)]).wait()

    def reduce_block(c):
        def red_g(g, _):
            rbase = c * _CHUNK + g * _LANES
            acc = jnp.zeros((_LANES,), jnp.float32)
            for f in range(_F):
                o = f * _BPW + rbase
                acc = acc + embv[pl.ds(o, _LANES)] * wv[pl.ds(o, _LANES)]
            outv[pl.ds(rbase, _LANES)] = acc
            return 0
        lax.fori_loop(0, _CHUNK // _LANES, red_g, 0)

    nblk = _BPW // _CHUNK
    _AHEAD = 4
    for c in range(_AHEAD):
        idx_block(c)
        fire_block(c)
    for cp in wcps:
        cp.wait()
    for c in range(_AHEAD, nblk):
        idx_block(c)
        fire_block(c)
        drain_block(c - _AHEAD)
        reduce_block(c - _AHEAD)
    for c in range(nblk - _AHEAD, nblk):
        drain_block(c)
        reduce_block(c)

    pltpu.sync_copy(outv, out_hbm.at[pl.ds(bbase, _BPW)])


_MESH = plsc.VectorSubcoreMesh(core_axis_name="c", subcore_axis_name="s")


@jax.jit
def _sc_lookup(table2d, x2d, w_t):
    f = pl.kernel(
        _body,
        out_type=jax.ShapeDtypeStruct((_B,), jnp.float32),
        mesh=_MESH,
        scratch_types=[
            pltpu.VMEM((_F, _BPW), jnp.int32),
            pltpu.VMEM((_EPW,), jnp.float32),
            pltpu.VMEM((_EPW,), jnp.int32),
            pltpu.VMEM((_EPW,), jnp.float32),
            pltpu.VMEM((_BPW,), jnp.float32),
            pltpu.SemaphoreType.DMA,
            pltpu.SemaphoreType.DMA,
            pltpu.SemaphoreType.DMA,
            pltpu.SemaphoreType.DMA,
            pltpu.SemaphoreType.DMA,
            pltpu.SemaphoreType.DMA,
        ],
    )
    return f(table2d, x2d, w_t)


def kernel(x, weight, fc_table, bias):
    w_t = jnp.transpose(weight, (1, 2, 0)).reshape(-1)
    out = _sc_lookup(fc_table.T, x.T, w_t)
    return out[:, None] + bias[None, :]
